# baseline (speedup 1.0000x reference)
import functools

import jax
import jax.numpy as jnp
from jax import lax
from jax.experimental import pallas as pl
from jax.experimental.pallas import tpu as pltpu
from jax.experimental.pallas import tpu_sc as plsc

_NC = 2
_NS = 16
_NW = _NC * _NS

_CHUNK = 80


def _transform_body(e_ref, w_ref, b_ref, o_ref):
    o_ref[...] = lax.dot_general(
        e_ref[...], w_ref[...],
        dimension_numbers=(((1,), (1,)), ((), ())),
        preferred_element_type=jnp.float32,
    ) + b_ref[...]


def _transform_table(embed_table, fc_w, fc_b):
    vocab, dim = embed_table.shape
    block = 2000
    grid = vocab // block
    return pl.pallas_call(
        _transform_body,
        grid=(grid,),
        in_specs=[
            pl.BlockSpec((block, dim), lambda i: (i, 0)),
            pl.BlockSpec((dim, dim), lambda i: (0, 0)),
            pl.BlockSpec((1, dim), lambda i: (0, 0)),
        ],
        out_specs=pl.BlockSpec((block, dim), lambda i: (i, 0)),
        out_shape=jax.ShapeDtypeStruct((vocab, dim), jnp.float32),
    )(embed_table, fc_w, fc_b.reshape(1, dim))


def _make_gather(n_ids, dim):
    assert n_ids % (_NW * 2 * _CHUNK) == 0
    b_per_w = n_ids // _NW
    n_chunks = b_per_w // _CHUNK
    n_pairs = n_chunks // 2
    mesh = plsc.VectorSubcoreMesh(core_axis_name="c", subcore_axis_name="s")

    @functools.partial(
        pl.kernel,
        mesh=mesh,
        out_type=jax.ShapeDtypeStruct((n_ids, dim // 2), jnp.int32),
        scratch_types=[
            pltpu.VMEM((b_per_w,), jnp.int32),
            pltpu.VMEM((_CHUNK, dim // 2), jnp.int32),
            pltpu.VMEM((_CHUNK, dim // 2), jnp.int32),
            pltpu.SemaphoreType.DMA,
            pltpu.SemaphoreType.DMA,
        ],
    )
    def gather_kernel(ids_hbm, table_hbm, out_hbm, idx_v, rows_a, rows_b,
                      sem_a, sem_b):
        wid = lax.axis_index("s") * _NC + lax.axis_index("c")
        base = wid * b_per_w
        pltpu.sync_copy(ids_hbm.at[pl.ds(base, b_per_w)], idx_v)

        def idx_at(j):
            return idx_v.at[pl.ds(j * _CHUNK, _CHUNK)]

        pltpu.async_copy(table_hbm.at[idx_at(0)], rows_a, sem_a)

        def pair(t, carry):
            j0 = 2 * t
            pltpu.make_async_copy(table_hbm.at[idx_at(j0)], rows_a, sem_a).wait()
            pltpu.async_copy(table_hbm.at[idx_at(j0 + 1)], rows_b, sem_b)
            pltpu.sync_copy(rows_a, out_hbm.at[pl.ds(base + j0 * _CHUNK, _CHUNK)])
            pltpu.make_async_copy(table_hbm.at[idx_at(j0 + 1)], rows_b, sem_b).wait()

            @pl.when(t + 1 < n_pairs)
            def _():
                pltpu.async_copy(table_hbm.at[idx_at(j0 + 2)], rows_a, sem_a)

            pltpu.sync_copy(rows_b, out_hbm.at[pl.ds(base + (j0 + 1) * _CHUNK, _CHUNK)])
            return carry

        lax.fori_loop(0, n_pairs, pair, 0)

    return gather_kernel


def kernel(input_ids, embed_table, fc_w, fc_b):
    b, l = input_ids.shape
    vocab, dim = embed_table.shape
    table_bf = _transform_table(embed_table, fc_w, fc_b).astype(jnp.bfloat16)
    table_t = lax.bitcast_convert_type(
        table_bf.reshape(vocab, dim // 2, 2), jnp.int32)
    ids_flat = input_ids.reshape(-1).astype(jnp.int32)
    out_flat = _make_gather(b * l, dim)(ids_flat, table_t)
    half = lax.bitcast_convert_type(out_flat, jnp.float32)
    return jnp.concatenate([half, half], axis=-1).reshape(b, l, dim)


# speedup vs baseline: 2.6634x; 2.6634x over previous
import functools

import jax
import jax.numpy as jnp
from jax import lax
from jax.experimental import pallas as pl
from jax.experimental.pallas import tpu as pltpu
from jax.experimental.pallas import tpu_sc as plsc

_NC = 2
_NS = 16
_NW = _NC * _NS

_CHUNK = 80


def _transform_body(e_ref, w_ref, b_ref, o_ref):
    o_ref[...] = lax.dot_general(
        e_ref[...].astype(jnp.bfloat16), w_ref[...].astype(jnp.bfloat16),
        dimension_numbers=(((1,), (1,)), ((), ())),
        preferred_element_type=jnp.float32,
    ) + b_ref[...]


def _transform_table(embed_table, fc_w, fc_b):
    vocab, dim = embed_table.shape
    block = 2000
    grid = vocab // block
    return pl.pallas_call(
        _transform_body,
        grid=(grid,),
        in_specs=[
            pl.BlockSpec((block, dim), lambda i: (i, 0)),
            pl.BlockSpec((dim, dim), lambda i: (0, 0)),
            pl.BlockSpec((1, dim), lambda i: (0, 0)),
        ],
        out_specs=pl.BlockSpec((block, dim), lambda i: (i, 0)),
        out_shape=jax.ShapeDtypeStruct((vocab, dim), jnp.float32),
    )(embed_table, fc_w, fc_b.reshape(1, dim))


def _make_gather(n_ids, dim):
    assert n_ids % (_NW * 2 * _CHUNK) == 0
    b_per_w = n_ids // _NW
    n_chunks = b_per_w // _CHUNK
    n_pairs = n_chunks // 2
    mesh = plsc.VectorSubcoreMesh(core_axis_name="c", subcore_axis_name="s")

    @functools.partial(
        pl.kernel,
        mesh=mesh,
        out_type=jax.ShapeDtypeStruct((n_ids, dim), jnp.float32),
        scratch_types=[
            pltpu.VMEM((b_per_w,), jnp.int32),
            pltpu.VMEM((_CHUNK, dim), jnp.float32),
            pltpu.VMEM((_CHUNK, dim), jnp.float32),
            pltpu.SemaphoreType.DMA,
            pltpu.SemaphoreType.DMA,
        ],
    )
    def gather_kernel(ids_hbm, table_hbm, out_hbm, idx_v, rows_a, rows_b,
                      sem_a, sem_b):
        wid = lax.axis_index("s") * _NC + lax.axis_index("c")
        base = wid * b_per_w
        pltpu.sync_copy(ids_hbm.at[pl.ds(base, b_per_w)], idx_v)

        def idx_at(j):
            return idx_v.at[pl.ds(j * _CHUNK, _CHUNK)]

        pltpu.async_copy(table_hbm.at[idx_at(0)], rows_a, sem_a)

        def pair(t, carry):
            j0 = 2 * t
            pltpu.make_async_copy(table_hbm.at[idx_at(j0)], rows_a, sem_a).wait()
            pltpu.async_copy(table_hbm.at[idx_at(j0 + 1)], rows_b, sem_b)
            pltpu.sync_copy(rows_a, out_hbm.at[pl.ds(base + j0 * _CHUNK, _CHUNK)])
            pltpu.make_async_copy(table_hbm.at[idx_at(j0 + 1)], rows_b, sem_b).wait()

            @pl.when(t + 1 < n_pairs)
            def _():
                pltpu.async_copy(table_hbm.at[idx_at(j0 + 2)], rows_a, sem_a)

            pltpu.sync_copy(rows_b, out_hbm.at[pl.ds(base + (j0 + 1) * _CHUNK, _CHUNK)])
            return carry

        lax.fori_loop(0, n_pairs, pair, 0)

    return gather_kernel


def kernel(input_ids, embed_table, fc_w, fc_b):
    b, l = input_ids.shape
    vocab, dim = embed_table.shape
    table_t = _transform_table(embed_table, fc_w, fc_b)
    ids_flat = input_ids.reshape(-1).astype(jnp.int32)
    out_flat = _make_gather(b * l, dim)(ids_flat, table_t)
    return out_flat.reshape(b, l, dim)


# speedup vs baseline: 2.6698x; 1.0024x over previous
import functools

import jax
import jax.numpy as jnp
from jax import lax
from jax.experimental import pallas as pl
from jax.experimental.pallas import tpu as pltpu
from jax.experimental.pallas import tpu_sc as plsc

_NC = 2
_NS = 16
_NW = _NC * _NS

_CHUNK = 80


def _transform_body(e_ref, w_ref, b_ref, o_ref):
    o_ref[...] = lax.dot_general(
        e_ref[...], w_ref[...],
        dimension_numbers=(((1,), (1,)), ((), ())),
        preferred_element_type=jnp.float32,
    ) + b_ref[...]


def _transform_table(embed_table, fc_w, fc_b):
    vocab, dim = embed_table.shape
    block = 3000
    grid = vocab // block
    return pl.pallas_call(
        _transform_body,
        grid=(grid,),
        in_specs=[
            pl.BlockSpec((block, dim), lambda i: (i, 0)),
            pl.BlockSpec((dim, dim), lambda i: (0, 0)),
            pl.BlockSpec((1, dim), lambda i: (0, 0)),
        ],
        out_specs=pl.BlockSpec((block, dim), lambda i: (i, 0)),
        out_shape=jax.ShapeDtypeStruct((vocab, dim), jnp.float32),
    )(embed_table, fc_w, fc_b.reshape(1, dim))


def _make_gather(n_ids, dim):
    assert n_ids % (_NW * 2 * _CHUNK) == 0
    b_per_w = n_ids // _NW
    n_chunks = b_per_w // _CHUNK
    n_pairs = n_chunks // 2
    mesh = plsc.VectorSubcoreMesh(core_axis_name="c", subcore_axis_name="s")

    @functools.partial(
        pl.kernel,
        mesh=mesh,
        out_type=jax.ShapeDtypeStruct((n_ids, dim), jnp.float32),
        scratch_types=[
            pltpu.VMEM((b_per_w,), jnp.int32),
            pltpu.VMEM((_CHUNK, dim), jnp.float32),
            pltpu.VMEM((_CHUNK, dim), jnp.float32),
            pltpu.SemaphoreType.DMA,
            pltpu.SemaphoreType.DMA,
        ],
    )
    def gather_kernel(ids_hbm, table_hbm, out_hbm, idx_v, rows_a, rows_b,
                      sem_a, sem_b):
        wid = lax.axis_index("s") * _NC + lax.axis_index("c")
        base = wid * b_per_w
        pltpu.sync_copy(ids_hbm.at[pl.ds(base, b_per_w)], idx_v)

        def idx_at(j):
            return idx_v.at[pl.ds(j * _CHUNK, _CHUNK)]

        pltpu.async_copy(table_hbm.at[idx_at(0)], rows_a, sem_a)

        def pair(t, carry):
            j0 = 2 * t
            pltpu.make_async_copy(table_hbm.at[idx_at(j0)], rows_a, sem_a).wait()
            pltpu.async_copy(table_hbm.at[idx_at(j0 + 1)], rows_b, sem_b)
            pltpu.sync_copy(rows_a, out_hbm.at[pl.ds(base + j0 * _CHUNK, _CHUNK)])
            pltpu.make_async_copy(table_hbm.at[idx_at(j0 + 1)], rows_b, sem_b).wait()

            @pl.when(t + 1 < n_pairs)
            def _():
                pltpu.async_copy(table_hbm.at[idx_at(j0 + 2)], rows_a, sem_a)

            pltpu.sync_copy(rows_b, out_hbm.at[pl.ds(base + (j0 + 1) * _CHUNK, _CHUNK)])
            return carry

        lax.fori_loop(0, n_pairs, pair, 0)

    return gather_kernel


def kernel(input_ids, embed_table, fc_w, fc_b):
    b, l = input_ids.shape
    vocab, dim = embed_table.shape
    table_t = _transform_table(embed_table, fc_w, fc_b)
    ids_flat = input_ids.reshape(-1).astype(jnp.int32)
    out_flat = _make_gather(b * l, dim)(ids_flat, table_t)
    return out_flat.reshape(b, l, dim)
